# baseline (speedup 1.0000x reference)
import functools

import jax
import jax.numpy as jnp
from jax import lax
from jax.experimental import pallas as pl
from jax.experimental.pallas import tpu as pltpu
from jax.experimental.pallas import tpu_sc as plsc

ROWS = 16384
COLS = 2048
L = 16
NC = 2
NS = 16
NW = NC * NS
RPW = ROWS // NW
CHUNK = 16
NCHUNK = RPW // CHUNK
JGROUPS = COLS // L


def _permute_body(x_hbm, perm_hbm, out_hbm, perm_v, in_v, out_v):
    wid = lax.axis_index("s") * NC + lax.axis_index("c")
    row0 = wid * RPW

    pltpu.sync_copy(perm_hbm, perm_v)

    def chunk_body(c, carry):
        base = (row0 + c * CHUNK) * COLS
        pltpu.sync_copy(x_hbm.at[pl.ds(base, CHUNK * COLS)], in_v)

        def col_group(j, carry2):
            colv = perm_v[pl.ds(j * L, L)]

            def row_step(r, carry3):
                idxv = colv + r * COLS
                vals = plsc.load_gather(in_v, [idxv])
                out_v[pl.ds(r * COLS + j * L, L)] = vals
                return carry3

            lax.fori_loop(0, CHUNK, row_step, 0)
            return carry2

        lax.fori_loop(0, JGROUPS, col_group, 0)
        pltpu.sync_copy(out_v, out_hbm.at[pl.ds(base, CHUNK * COLS)])
        return carry

    lax.fori_loop(0, NCHUNK, chunk_body, 0)


_mesh = plsc.VectorSubcoreMesh(core_axis_name="c", subcore_axis_name="s")

_permute = functools.partial(
    pl.kernel,
    mesh=_mesh,
    out_type=jax.ShapeDtypeStruct((ROWS * COLS,), jnp.float32),
    scratch_types=[
        pltpu.VMEM((COLS,), jnp.int32),
        pltpu.VMEM((CHUNK * COLS,), jnp.float32),
        pltpu.VMEM((CHUNK * COLS,), jnp.float32),
    ],
    compiler_params=pltpu.CompilerParams(needs_layout_passes=False),
)(_permute_body)


def kernel(x, perm):
    flat = _permute(x.reshape(-1), perm.astype(jnp.int32))
    return flat.reshape(ROWS, COLS)


# speedup vs baseline: 1.0030x; 1.0030x over previous
import functools

import jax
import jax.numpy as jnp
from jax import lax
from jax.experimental import pallas as pl
from jax.experimental.pallas import tpu as pltpu
from jax.experimental.pallas import tpu_sc as plsc

ROWS = 16384
COLS = 2048
L = 16
NC = 2
NS = 16
NW = NC * NS
RPW = ROWS // NW
CHUNK = 16
NCHUNK = RPW // CHUNK
JGROUPS = COLS // L


def _permute_body(x_hbm, perm_hbm, out_hbm, perm_v, in_v, out_v):
    wid = lax.axis_index("s") * NC + lax.axis_index("c")
    row0 = wid * RPW

    pltpu.sync_copy(perm_hbm, perm_v)

    def chunk_body(c, carry):
        base = (row0 + c * CHUNK) * COLS
        pltpu.sync_copy(x_hbm.at[pl.ds(base, CHUNK * COLS)], in_v)

        def col_group(j, carry2):
            colv = perm_v[pl.ds(j * L, L)]
            base_j = j * L
            for r in range(CHUNK):
                vals = plsc.load_gather(in_v, [colv + r * COLS])
                out_v[pl.ds(base_j + r * COLS, L)] = vals
            return carry2

        lax.fori_loop(0, JGROUPS, col_group, 0)
        pltpu.sync_copy(out_v, out_hbm.at[pl.ds(base, CHUNK * COLS)])
        return carry

    lax.fori_loop(0, NCHUNK, chunk_body, 0)


_mesh = plsc.VectorSubcoreMesh(core_axis_name="c", subcore_axis_name="s")

_permute = functools.partial(
    pl.kernel,
    mesh=_mesh,
    out_type=jax.ShapeDtypeStruct((ROWS * COLS,), jnp.float32),
    scratch_types=[
        pltpu.VMEM((COLS,), jnp.int32),
        pltpu.VMEM((CHUNK * COLS,), jnp.float32),
        pltpu.VMEM((CHUNK * COLS,), jnp.float32),
    ],
    compiler_params=pltpu.CompilerParams(needs_layout_passes=False),
)(_permute_body)


def kernel(x, perm):
    flat = _permute(x.reshape(-1), perm.astype(jnp.int32))
    return flat.reshape(ROWS, COLS)


# speedup vs baseline: 1.1460x; 1.1427x over previous
import functools

import jax
import jax.numpy as jnp
from jax import lax
from jax.experimental import pallas as pl
from jax.experimental.pallas import tpu as pltpu
from jax.experimental.pallas import tpu_sc as plsc

ROWS = 16384
COLS = 2048
L = 16
NC = 2
NS = 16
NW = NC * NS
RPW = ROWS // NW
CHUNK = 8
NCHUNK = RPW // CHUNK
CELEMS = CHUNK * COLS
JGROUPS = COLS // L


def _permute_body(x_hbm, perm_hbm, out_hbm, perm_v,
                  in_v0, in_v1, out_v0, out_v1,
                  isem0, isem1, osem0, osem1):
    wid = lax.axis_index("s") * NC + lax.axis_index("c")
    base0 = wid * RPW * COLS

    pltpu.sync_copy(perm_hbm, perm_v)

    def in_slice(c):
        return x_hbm.at[pl.ds(base0 + c * CELEMS, CELEMS)]

    def out_slice(c):
        return out_hbm.at[pl.ds(base0 + c * CELEMS, CELEMS)]

    def compute(in_v, out_v):
        def col_group(j, carry):
            colv = perm_v[pl.ds(j * L, L)]
            base_j = j * L
            for r in range(CHUNK):
                vals = plsc.load_gather(in_v, [colv + r * COLS])
                out_v[pl.ds(base_j + r * COLS, L)] = vals
            return carry

        lax.fori_loop(0, JGROUPS, col_group, 0)

    pltpu.async_copy(in_slice(0), in_v0, isem0)

    def step(t, carry):
        def run(in_v, out_v, isem_cur, isem_nxt, osem, in_nxt):
            @pl.when(t + 1 < NCHUNK)
            def _():
                pltpu.async_copy(in_slice(t + 1), in_nxt, isem_nxt)

            pltpu.make_async_copy(in_slice(t), in_v, isem_cur).wait()

            @pl.when(t >= 2)
            def _():
                pltpu.make_async_copy(out_v, out_slice(t - 2), osem).wait()

            compute(in_v, out_v)
            pltpu.async_copy(out_v, out_slice(t), osem)

        @pl.when(t % 2 == 0)
        def _():
            run(in_v0, out_v0, isem0, isem1, osem0, in_v1)

        @pl.when(t % 2 == 1)
        def _():
            run(in_v1, out_v1, isem1, isem0, osem1, in_v0)

        return carry

    lax.fori_loop(0, NCHUNK, step, 0)

    pltpu.make_async_copy(out_v0, out_slice(NCHUNK - 2), osem0).wait()
    pltpu.make_async_copy(out_v1, out_slice(NCHUNK - 1), osem1).wait()


_mesh = plsc.VectorSubcoreMesh(core_axis_name="c", subcore_axis_name="s")

_permute = functools.partial(
    pl.kernel,
    mesh=_mesh,
    out_type=jax.ShapeDtypeStruct((ROWS * COLS,), jnp.float32),
    scratch_types=[
        pltpu.VMEM((COLS,), jnp.int32),
        pltpu.VMEM((CELEMS,), jnp.float32),
        pltpu.VMEM((CELEMS,), jnp.float32),
        pltpu.VMEM((CELEMS,), jnp.float32),
        pltpu.VMEM((CELEMS,), jnp.float32),
        pltpu.SemaphoreType.DMA,
        pltpu.SemaphoreType.DMA,
        pltpu.SemaphoreType.DMA,
        pltpu.SemaphoreType.DMA,
    ],
    compiler_params=pltpu.CompilerParams(needs_layout_passes=False),
)(_permute_body)


def kernel(x, perm):
    flat = _permute(x.reshape(-1), perm.astype(jnp.int32))
    return flat.reshape(ROWS, COLS)
